# baseline (speedup 1.0000x reference)
import functools

import jax
import jax.numpy as jnp
from jax import lax
from jax.experimental import pallas as pl
from jax.experimental.pallas import tpu as pltpu

_NUM_EMBEDDINGS = 1024
_EMBEDDING_DIM = 64
_BETA = 0.25
_N_ROWS = 16 * 1024
_BLOCK = 1024
_GRID = _N_ROWS // _BLOCK


def _vq_block(x_ref, c_ref, ct_ref, q_ref, counts_ref, loss_ref, perp_ref):
    g = pl.program_id(0)

    x = x_ref[...]
    c = c_ref[...]
    ct = ct_ref[...]

    cc = jnp.sum(c * c, axis=0, keepdims=True)
    d = cc - 2.0 * jnp.dot(x, c, preferred_element_type=jnp.float32)

    dmin = jnp.min(d, axis=1, keepdims=True)
    onehot = (d == dmin).astype(jnp.float32)

    q = jnp.dot(onehot, ct, preferred_element_type=jnp.float32)
    q_ref[...] = q

    @pl.when(g == 0)
    def _init():
        counts_ref[...] = jnp.zeros_like(counts_ref)
        loss_ref[0, 0] = 0.0
        perp_ref[0, 0] = 0.0

    counts_ref[...] += jnp.sum(onehot, axis=0, keepdims=True)
    loss_ref[0, 0] += jnp.sum(dmin) + jnp.sum(x * x)

    @pl.when(g == _GRID - 1)
    def _finalize():
        loss_ref[0, 0] = loss_ref[0, 0] * (_BETA / (_N_ROWS * _EMBEDDING_DIM))
        p = counts_ref[...] * (1.0 / _N_ROWS)
        ent = -jnp.sum(p * jnp.log(p + 1e-10))
        perp_ref[0, 0] = jnp.exp(ent)


@jax.jit
def _vq_forward(flat_inputs, codebook, codebook_t):
    q, _counts, loss, perp = pl.pallas_call(
        _vq_block,
        grid=(_GRID,),
        in_specs=[
            pl.BlockSpec((_BLOCK, _EMBEDDING_DIM), lambda g: (g, 0)),
            pl.BlockSpec((_EMBEDDING_DIM, _NUM_EMBEDDINGS), lambda g: (0, 0)),
            pl.BlockSpec((_NUM_EMBEDDINGS, _EMBEDDING_DIM), lambda g: (0, 0)),
        ],
        out_specs=[
            pl.BlockSpec((_BLOCK, _EMBEDDING_DIM), lambda g: (g, 0)),
            pl.BlockSpec((1, _NUM_EMBEDDINGS), lambda g: (0, 0)),
            pl.BlockSpec(memory_space=pltpu.SMEM),
            pl.BlockSpec(memory_space=pltpu.SMEM),
        ],
        out_shape=[
            jax.ShapeDtypeStruct((_N_ROWS, _EMBEDDING_DIM), jnp.float32),
            jax.ShapeDtypeStruct((1, _NUM_EMBEDDINGS), jnp.float32),
            jax.ShapeDtypeStruct((1, 1), jnp.float32),
            jax.ShapeDtypeStruct((1, 1), jnp.float32),
        ],
    )(flat_inputs, codebook, codebook_t)
    return q, loss[0, 0], perp[0, 0]


def kernel(inputs, codebook, training=True):
    flat_inputs = jnp.reshape(inputs, (-1, _EMBEDDING_DIM))
    q, loss, perp = _vq_forward(flat_inputs, codebook, codebook.T)
    ste = jnp.reshape(q, inputs.shape)
    return ste, perp, loss


# speedup vs baseline: 1.0510x; 1.0510x over previous
import functools

import jax
import jax.numpy as jnp
from jax import lax
from jax.experimental import pallas as pl
from jax.experimental.pallas import tpu as pltpu

_NUM_EMBEDDINGS = 1024
_EMBEDDING_DIM = 64
_BETA = 0.25
_N_ROWS = 16 * 1024
_BLOCK = 2048
_GRID = _N_ROWS // _BLOCK


def _vq_block(x_ref, c_ref, ct_ref, q_ref, counts_ref, loss_ref, perp_ref):
    g = pl.program_id(0)

    x = x_ref[...]
    c = c_ref[...]
    ct = ct_ref[...]

    cc = jnp.sum(c * c, axis=0, keepdims=True)
    d = cc - 2.0 * jnp.dot(x, c, preferred_element_type=jnp.float32)

    dmin = jnp.min(d, axis=1, keepdims=True)
    onehot = (d == dmin).astype(jnp.float32)

    q = jnp.dot(onehot, ct, preferred_element_type=jnp.float32)
    q_ref[...] = q

    @pl.when(g == 0)
    def _init():
        counts_ref[...] = jnp.zeros_like(counts_ref)
        loss_ref[0, 0] = 0.0
        perp_ref[0, 0] = 0.0

    counts_ref[...] += jnp.sum(onehot, axis=0, keepdims=True)
    loss_ref[0, 0] += jnp.sum(dmin) + jnp.sum(x * x)

    @pl.when(g == _GRID - 1)
    def _finalize():
        loss_ref[0, 0] = loss_ref[0, 0] * (_BETA / (_N_ROWS * _EMBEDDING_DIM))
        p = counts_ref[...] * (1.0 / _N_ROWS)
        ent = -jnp.sum(p * jnp.log(p + 1e-10))
        perp_ref[0, 0] = jnp.exp(ent)


@jax.jit
def _vq_forward(flat_inputs, codebook, codebook_t):
    q, _counts, loss, perp = pl.pallas_call(
        _vq_block,
        grid=(_GRID,),
        in_specs=[
            pl.BlockSpec((_BLOCK, _EMBEDDING_DIM), lambda g: (g, 0)),
            pl.BlockSpec((_EMBEDDING_DIM, _NUM_EMBEDDINGS), lambda g: (0, 0)),
            pl.BlockSpec((_NUM_EMBEDDINGS, _EMBEDDING_DIM), lambda g: (0, 0)),
        ],
        out_specs=[
            pl.BlockSpec((_BLOCK, _EMBEDDING_DIM), lambda g: (g, 0)),
            pl.BlockSpec((1, _NUM_EMBEDDINGS), lambda g: (0, 0)),
            pl.BlockSpec(memory_space=pltpu.SMEM),
            pl.BlockSpec(memory_space=pltpu.SMEM),
        ],
        out_shape=[
            jax.ShapeDtypeStruct((_N_ROWS, _EMBEDDING_DIM), jnp.float32),
            jax.ShapeDtypeStruct((1, _NUM_EMBEDDINGS), jnp.float32),
            jax.ShapeDtypeStruct((1, 1), jnp.float32),
            jax.ShapeDtypeStruct((1, 1), jnp.float32),
        ],
    )(flat_inputs, codebook, codebook_t)
    return q, loss[0, 0], perp[0, 0]


def kernel(inputs, codebook, training=True):
    flat_inputs = jnp.reshape(inputs, (-1, _EMBEDDING_DIM))
    q, loss, perp = _vq_forward(flat_inputs, codebook, codebook.T)
    ste = jnp.reshape(q, inputs.shape)
    return ste, perp, loss
